# Initial kernel scaffold; baseline (speedup 1.0000x reference)
import functools

import jax
import jax.numpy as jnp
from jax import lax
from jax.experimental import pallas as pl
from jax.experimental.pallas import tpu as pltpu
from jax.experimental.pallas import tpu_sc as plsc

N = 200000
C = 128
LOCAL_SIZE = 256
SCALE_SHIFT = 2
GLOBAL_SIZE = 64

NC = 2
NS = 16
NW = NC * NS

BLK = 400
NBLK = N // BLK
IDX_CHUNK = 80
N_CHUNKS = BLK // IDX_CHUNK
LANES = 16
IDX_STEPS = BLK // LANES


def _fusion_body(xs_hbm, ys_hbm, zs_hbm, lf_hbm, gf_hbm, base_hbm, out_hbm,
                 xv, yv, zv, basev, lbuf, gbuf,
                 idx0, idx1, idx2, idx3, idx4,
                 sem_l, sem_g):
    idx_bufs = (idx0, idx1, idx2, idx3, idx4)
    wid = lax.axis_index("s") * NC + lax.axis_index("c")

    pltpu.sync_copy(base_hbm, basev)
    lb0 = basev[0]
    lb1 = basev[1]
    lb2 = basev[2]
    gb0 = basev[3]
    gb1 = basev[4]
    gb2 = basev[5]

    nblocks_w = (NBLK - wid + NW - 1) // NW

    def block_body(i, carry):
        b = wid + i * NW
        base = b * BLK

        cp_l = pltpu.async_copy(lf_hbm.at[pl.ds(base, BLK)], lbuf, sem_l)

        pltpu.sync_copy(xs_hbm.at[pl.ds(base, BLK)], xv)
        pltpu.sync_copy(ys_hbm.at[pl.ds(base, BLK)], yv)
        pltpu.sync_copy(zs_hbm.at[pl.ds(base, BLK)], zv)

        for s in range(IDX_STEPS):
            sl = pl.ds(s * LANES, LANES)
            gx = jnp.clip(((xv[sl] + lb0) >> SCALE_SHIFT) + gb0, 0, GLOBAL_SIZE - 1)
            gy = jnp.clip(((yv[sl] + lb1) >> SCALE_SHIFT) + gb1, 0, GLOBAL_SIZE - 1)
            gz = jnp.clip(((zv[sl] + lb2) >> SCALE_SHIFT) + gb2, 0, GLOBAL_SIZE - 1)
            flat = (gx * (GLOBAL_SIZE * GLOBAL_SIZE)) + (gy * GLOBAL_SIZE) + gz
            j, col = divmod(s * LANES, IDX_CHUNK)
            idx_bufs[j][pl.ds(col, LANES)] = flat

        cps = [
            pltpu.async_copy(
                gf_hbm.at[idx_bufs[j]],
                gbuf.at[pl.ds(j * IDX_CHUNK, IDX_CHUNK)],
                sem_g,
            )
            for j in range(N_CHUNKS)
        ]
        cp_l.wait()
        for cp in cps:
            cp.wait()

        def row_add(r, c2):
            for col in range(C // LANES):
                sl = pl.ds(col * LANES, LANES)
                plsc.addupdate(lbuf.at[r, sl], gbuf[r, sl])
            return c2

        lax.fori_loop(0, BLK, row_add, 0)

        pltpu.sync_copy(lbuf, out_hbm.at[pl.ds(base, BLK)])
        return carry

    lax.fori_loop(0, nblocks_w, block_body, 0)


@jax.jit
def _fusion(xs, ys, zs, lf, gf, base16):
    mesh = plsc.VectorSubcoreMesh(core_axis_name="c", subcore_axis_name="s")
    return pl.kernel(
        _fusion_body,
        mesh=mesh,
        out_type=jax.ShapeDtypeStruct((N, C), jnp.float32),
        scratch_types=[
            pltpu.VMEM((BLK,), jnp.int32),
            pltpu.VMEM((BLK,), jnp.int32),
            pltpu.VMEM((BLK,), jnp.int32),
            pltpu.VMEM((16,), jnp.int32),
            pltpu.VMEM((BLK, C), jnp.float32),
            pltpu.VMEM((BLK, C), jnp.float32),
            pltpu.VMEM((IDX_CHUNK,), jnp.int32),
            pltpu.VMEM((IDX_CHUNK,), jnp.int32),
            pltpu.VMEM((IDX_CHUNK,), jnp.int32),
            pltpu.VMEM((IDX_CHUNK,), jnp.int32),
            pltpu.VMEM((IDX_CHUNK,), jnp.int32),
            pltpu.SemaphoreType.DMA,
            pltpu.SemaphoreType.DMA,
        ],
    )(xs, ys, zs, lf, gf, base16)


def kernel(local_features, local_coords, global_features, local_base, global_base):
    coords = local_coords.astype(jnp.int32)
    xs = coords[:, 0]
    ys = coords[:, 1]
    zs = coords[:, 2]
    base16 = jnp.zeros((16,), jnp.int32)
    base16 = base16.at[0:3].set(local_base.astype(jnp.int32))
    base16 = base16.at[3:6].set(global_base.astype(jnp.int32))
    return _fusion(xs, ys, zs, local_features, global_features, base16)


# speedup vs baseline: 1.2933x; 1.2933x over previous
import jax
import jax.numpy as jnp
from jax import lax
from jax.experimental import pallas as pl
from jax.experimental.pallas import tpu as pltpu
from jax.experimental.pallas import tpu_sc as plsc

N = 200000
C = 128
SCALE_SHIFT = 2
GLOBAL_SIZE = 64

NC = 2
NS = 16
NW = NC * NS

BLK = 160
NBLK = N // BLK
IDX_CHUNK = 80
N_CHUNKS = BLK // IDX_CHUNK
LANES = 16
IDX_STEPS = BLK // LANES
NSLOT = 3
NB_MAX = (NBLK + NW - 1) // NW
UB = (NB_MAX + NSLOT - 1) // NSLOT


def _fusion_body(co_hbm, lf_hbm, gf_hbm, base_hbm, out_hbm,
                 cob0, cob1, cob2, lb0_, lb1_, lb2_, gb0_, gb1_, gb2_,
                 ix00, ix01, ix10, ix11, ix20, ix21, basev,
                 sc0, sc1, sc2, sl0, sl1, sl2, sg00, sg01, sg10, sg11,
                 sg20, sg21, so0, so1, so2):
    cob = (cob0, cob1, cob2)
    lbuf = (lb0_, lb1_, lb2_)
    gbuf = (gb0_, gb1_, gb2_)
    ixb = ((ix00, ix01), (ix10, ix11), (ix20, ix21))
    sem_c = (sc0, sc1, sc2)
    sem_l = (sl0, sl1, sl2)
    sem_g = ((sg00, sg01), (sg10, sg11), (sg20, sg21))
    sem_o = (so0, so1, so2)

    wid = lax.axis_index("s") * NC + lax.axis_index("c")
    nb_w = (NBLK - wid + NW - 1) // NW

    pltpu.sync_copy(base_hbm, basev)
    bvec = basev[pl.ds(0, 16)]
    lb = (bvec[0], bvec[1], bvec[2])
    gb = (bvec[3], bvec[4], bvec[5])

    def fire_inputs(s, k):
        b = wid + k * NW
        pltpu.async_copy(co_hbm.at[b], cob[s], sem_c[s])
        pltpu.async_copy(lf_hbm.at[pl.ds(b * BLK, BLK)], lbuf[s], sem_l[s])

    def wait_coords(s, k):
        b = wid + k * NW
        pltpu.make_async_copy(co_hbm.at[b], cob[s], sem_c[s]).wait()

    def wait_local(s, k):
        b = wid + k * NW
        pltpu.make_async_copy(
            lf_hbm.at[pl.ds(b * BLK, BLK)], lbuf[s], sem_l[s]).wait()

    def wait_out(s, k):
        b = wid + k * NW
        pltpu.make_async_copy(
            lbuf[s], out_hbm.at[pl.ds(b * BLK, BLK)], sem_o[s]).wait()

    def idx_and_gather(s):
        for st in range(IDX_STEPS):
            gx = jnp.clip(((cob[s][pl.ds(st * LANES, LANES)] + lb[0])
                           >> SCALE_SHIFT) + gb[0], 0, GLOBAL_SIZE - 1)
            gy = jnp.clip(((cob[s][pl.ds(BLK + st * LANES, LANES)] + lb[1])
                           >> SCALE_SHIFT) + gb[1], 0, GLOBAL_SIZE - 1)
            gz = jnp.clip(((cob[s][pl.ds(2 * BLK + st * LANES, LANES)] + lb[2])
                           >> SCALE_SHIFT) + gb[2], 0, GLOBAL_SIZE - 1)
            flat = gx * (GLOBAL_SIZE * GLOBAL_SIZE) + gy * GLOBAL_SIZE + gz
            j, col = divmod(st * LANES, IDX_CHUNK)
            ixb[s][j][pl.ds(col, LANES)] = flat
        for j in range(N_CHUNKS):
            pltpu.async_copy(
                gf_hbm.at[ixb[s][j]],
                gbuf[s].at[pl.ds(j * IDX_CHUNK, IDX_CHUNK)],
                sem_g[s][j],
            )

    def process(s, k):
        b = wid + k * NW
        wait_local(s, k)
        for j in range(N_CHUNKS):
            pltpu.make_async_copy(
                gf_hbm.at[ixb[s][j]],
                gbuf[s].at[pl.ds(j * IDX_CHUNK, IDX_CHUNK)],
                sem_g[s][j],
            ).wait()

            def row_add(r, acc):
                for col in range(C // LANES):
                    sl = pl.ds(col * LANES, LANES)
                    plsc.addupdate(lbuf[s].at[r, sl], gbuf[s][r, sl])
                return acc

            lax.fori_loop(j * IDX_CHUNK, (j + 1) * IDX_CHUNK, row_add, 0)
        pltpu.async_copy(lbuf[s], out_hbm.at[pl.ds(b * BLK, BLK)], sem_o[s])

    fire_inputs(0, 0)
    fire_inputs(1, 1)
    wait_coords(0, 0)
    idx_and_gather(0)

    def body(i, carry):
        for j in range(NSLOT):
            k = i * NSLOT + j
            s = j
            s1 = (j + 1) % NSLOT
            s2 = (j + 2) % NSLOT

            @pl.when(k < nb_w)
            def _():
                process(s, k)

            @pl.when(k + 2 < nb_w)
            def _():
                @pl.when(k >= 1)
                def _():
                    wait_out(s2, k - 1)
                fire_inputs(s2, k + 2)

            @pl.when(k + 1 < nb_w)
            def _():
                wait_coords(s1, k + 1)
                idx_and_gather(s1)
        return carry

    lax.fori_loop(0, UB, body, 0)

    for s in range(NSLOT):
        pltpu.make_async_copy(
            lbuf[s], out_hbm.at[pl.ds(0, BLK)], sem_o[s]).wait()


@jax.jit
def _fusion(co, lf, gf, base16):
    mesh = plsc.VectorSubcoreMesh(core_axis_name="c", subcore_axis_name="s")
    semt = pltpu.SemaphoreType.DMA
    return pl.kernel(
        _fusion_body,
        mesh=mesh,
        out_type=jax.ShapeDtypeStruct((N, C), jnp.float32),
        scratch_types=(
            [pltpu.VMEM((3 * BLK,), jnp.int32) for _ in range(3)]
            + [pltpu.VMEM((BLK, C), jnp.float32) for _ in range(6)]
            + [pltpu.VMEM((IDX_CHUNK,), jnp.int32) for _ in range(6)]
            + [pltpu.VMEM((16,), jnp.int32)]
            + [semt] * 15
        ),
    )(co, lf, gf, base16)


def kernel(local_features, local_coords, global_features, local_base, global_base):
    coords = local_coords.astype(jnp.int32)
    co = coords.reshape(NBLK, BLK, 3).transpose(0, 2, 1).reshape(NBLK, 3 * BLK)
    base16 = jnp.zeros((16,), jnp.int32)
    base16 = base16.at[0:3].set(local_base.astype(jnp.int32))
    base16 = base16.at[3:6].set(global_base.astype(jnp.int32))
    return _fusion(co, local_features, global_features, base16)
